# baseline (speedup 1.0000x reference)
import functools

import jax
import jax.numpy as jnp
from jax import lax
from jax.experimental import pallas as pl
from jax.experimental.pallas import tpu as pltpu
from jax.experimental.pallas import tpu_sc as plsc

N = 10000
E = 320000
D = 128
H = 128
OUT = 128

NC = 2
NS = 16
NW = NC * NS
CH = 128
NCHUNK = 80
HALF = NCHUNK // 2
PAIRS = HALF // 2
EPAD = NW * NCHUNK * CH
NPAD = 10240
RPT = NPAD // NS
CW = 16

BM = 1280
GRID = NPAD // BM

_mesh = plsc.VectorSubcoreMesh(core_axis_name="c", subcore_axis_name="s")


@functools.partial(
    pl.kernel,
    out_type=jax.ShapeDtypeStruct((NC, NPAD, D), jnp.float32),
    mesh=_mesh,
    scratch_types=[
        pltpu.VMEM((NCHUNK, CH), jnp.int32),
        pltpu.VMEM((CH, D), jnp.float32),
        pltpu.VMEM_SHARED((NPAD, D), jnp.float32),
        pltpu.SemaphoreType.DMA,
        pltpu.SemaphoreType.DMA,
    ],
)
def _count(dst3, zc, ones, c_out, dst_v, ones_v, cacc, sem0, sem1):
    cid = lax.axis_index("c")
    sid = lax.axis_index("s")
    wid = sid * NC + cid
    r0 = sid * RPT
    pltpu.sync_copy(zc.at[pl.ds(r0, RPT)], cacc.at[pl.ds(r0, RPT)])
    pltpu.sync_copy(ones, ones_v)
    pltpu.sync_copy(dst3.at[wid], dst_v)
    plsc.subcore_barrier()

    def step(g, carry):
        j0 = 2 * g
        d0 = pltpu.async_copy(ones_v, cacc.at[dst_v.at[j0]], sem0, add=True)
        d1 = pltpu.async_copy(ones_v, cacc.at[dst_v.at[j0 + 1]], sem1,
                              add=True)
        d0.wait()
        d1.wait()
        return carry

    lax.fori_loop(0, NCHUNK // 2, step, 0)
    plsc.subcore_barrier()
    pltpu.sync_copy(cacc.at[pl.ds(r0, RPT)], c_out.at[cid, pl.ds(r0, RPT)])


@functools.partial(
    pl.kernel,
    out_type=jax.ShapeDtypeStruct((NC, NPAD, D), jnp.float32),
    mesh=_mesh,
    scratch_types=[
        pltpu.VMEM((HALF, CH), jnp.int32),
        pltpu.VMEM((HALF, CH), jnp.int32),
        pltpu.VMEM((CH, D), jnp.float32),
        pltpu.VMEM((CH, D), jnp.float32),
        pltpu.VMEM_SHARED((NPAD, D), jnp.float32),
        pltpu.SemaphoreType.DMA,
        pltpu.SemaphoreType.DMA,
        pltpu.SemaphoreType.DMA,
        pltpu.SemaphoreType.DMA,
    ],
)
def _agg(h, src3, dst3, zh, p_out, src_v, dst_v, rows0, rows1, acc,
         gsem0, gsem1, ssem0, ssem1):
    cid = lax.axis_index("c")
    sid = lax.axis_index("s")
    wid = sid * NC + cid
    r0 = sid * RPT
    pltpu.sync_copy(zh.at[pl.ds(r0, RPT)], acc.at[pl.ds(r0, RPT)])
    plsc.subcore_barrier()

    for half in range(2):
        pltpu.sync_copy(src3.at[wid, pl.ds(half * HALF, HALF)], src_v)
        pltpu.sync_copy(dst3.at[wid, pl.ds(half * HALF, HALF)], dst_v)
        pltpu.async_copy(h.at[src_v.at[0]], rows0, gsem0)
        pltpu.async_copy(h.at[src_v.at[1]], rows1, gsem1)

        def pair(g, carry):
            j0 = 2 * g
            jn0 = jnp.minimum(j0 + 2, HALF - 1)
            jn1 = jnp.minimum(j0 + 3, HALF - 1)
            pltpu.make_async_copy(zh.at[pl.ds(0, CH)], rows0, gsem0).wait()
            s0 = pltpu.async_copy(rows0, acc.at[dst_v.at[j0]], ssem0,
                                  priority=1, add=True)
            pltpu.make_async_copy(zh.at[pl.ds(0, CH)], rows1, gsem1).wait()
            s1 = pltpu.async_copy(rows1, acc.at[dst_v.at[j0 + 1]], ssem1,
                                  priority=1, add=True)
            s0.wait()
            pltpu.async_copy(h.at[src_v.at[jn0]], rows0, gsem0)
            s1.wait()
            pltpu.async_copy(h.at[src_v.at[jn1]], rows1, gsem1)
            return carry

        lax.fori_loop(0, PAIRS, pair, 0)
        pltpu.make_async_copy(zh.at[pl.ds(0, CH)], rows0, gsem0).wait()
        pltpu.make_async_copy(zh.at[pl.ds(0, CH)], rows1, gsem1).wait()

    plsc.subcore_barrier()
    pltpu.sync_copy(acc.at[pl.ds(r0, RPT)], p_out.at[cid, pl.ds(r0, RPT)])


def _in_proj_k(x_ref, w_ref, b_ref, o_ref):
    o_ref[...] = jnp.maximum(
        jnp.dot(x_ref[...], w_ref[...], preferred_element_type=jnp.float32)
        + b_ref[...], 0.0)


def _comb_first_k(p_ref, c_ref, h_ref, r_ref):
    cnt = c_ref[0] + c_ref[1]
    r = 1.0 / jnp.maximum(cnt, 1.0)
    r_ref[...] = r[:, 0:CW]
    h_ref[...] = (p_ref[0] + p_ref[1]) * r[:, 0:1]


def _comb_k(p_ref, r_ref, h_ref):
    h_ref[...] = (p_ref[0] + p_ref[1]) * r_ref[:, 0:1]


def _mix_relu_k(h1_ref, p_ref, r_ref, wa_ref, wb_ref, b_ref, o_ref):
    h2 = (p_ref[0] + p_ref[1]) * r_ref[:, 0:1]
    acc = (jnp.dot(h1_ref[...], wa_ref[...], preferred_element_type=jnp.float32)
           + jnp.dot(h2, wb_ref[...], preferred_element_type=jnp.float32)
           + b_ref[...])
    o_ref[...] = jnp.maximum(acc, 0.0)


def _mix_out_k(h1_ref, p_ref, r_ref, wa_ref, wb_ref, b_ref, wo_ref, bo_ref,
               o_ref):
    h2 = (p_ref[0] + p_ref[1]) * r_ref[:, 0:1]
    t = (jnp.dot(h1_ref[...], wa_ref[...], preferred_element_type=jnp.float32)
         + jnp.dot(h2, wb_ref[...], preferred_element_type=jnp.float32)
         + b_ref[...])
    o_ref[...] = (jnp.dot(t, wo_ref[...], preferred_element_type=jnp.float32)
                  + bo_ref[...])


def _rows(i):
    return (i, 0)


def _full2(i):
    return (0, 0)


def _part3(i):
    return (0, i, 0)


_ROWS_D = pl.BlockSpec((BM, D), _rows)
_ROWS_CW = pl.BlockSpec((BM, CW), _rows)
_PART_D = pl.BlockSpec((NC, BM, D), _part3)
_PART_CW = pl.BlockSpec((NC, BM, CW), _part3)
_W_SPEC = pl.BlockSpec((D, H), _full2)
_B_SPEC = pl.BlockSpec((1, H), _full2)


def _in_proj(xp, W, b):
    return pl.pallas_call(
        _in_proj_k,
        grid=(GRID,),
        in_specs=[_ROWS_D, _W_SPEC, _B_SPEC],
        out_specs=_ROWS_D,
        out_shape=jax.ShapeDtypeStruct((NPAD, H), jnp.float32),
    )(xp, W, b.reshape(1, H))


def _comb_first(p, c):
    return pl.pallas_call(
        _comb_first_k,
        grid=(GRID,),
        in_specs=[_PART_D, _PART_D],
        out_specs=[_ROWS_D, _ROWS_CW],
        out_shape=[jax.ShapeDtypeStruct((NPAD, D), jnp.float32),
                   jax.ShapeDtypeStruct((NPAD, CW), jnp.float32)],
    )(p, c)


def _comb(p, recip):
    return pl.pallas_call(
        _comb_k,
        grid=(GRID,),
        in_specs=[_PART_D, _ROWS_CW],
        out_specs=_ROWS_D,
        out_shape=jax.ShapeDtypeStruct((NPAD, D), jnp.float32),
    )(p, recip)


def _mix_relu(h1, p, recip, Wa, Wb, b):
    return pl.pallas_call(
        _mix_relu_k,
        grid=(GRID,),
        in_specs=[_ROWS_D, _PART_D, _ROWS_CW, _W_SPEC, _W_SPEC, _B_SPEC],
        out_specs=_ROWS_D,
        out_shape=jax.ShapeDtypeStruct((NPAD, H), jnp.float32),
    )(h1, p, recip, Wa, Wb, b.reshape(1, H))


def _mix_out(h1, p, recip, Wa, Wb, b, Wo, bo):
    return pl.pallas_call(
        _mix_out_k,
        grid=(GRID,),
        in_specs=[_ROWS_D, _PART_D, _ROWS_CW, _W_SPEC, _W_SPEC, _B_SPEC,
                  _W_SPEC, _B_SPEC],
        out_specs=_ROWS_D,
        out_shape=jax.ShapeDtypeStruct((NPAD, OUT), jnp.float32),
    )(h1, p, recip, Wa, Wb, b.reshape(1, H), Wo, bo.reshape(1, OUT))


def kernel(x, edge_index, W_in, b_in, W_mix0, b_mix0, W_mix1, b_mix1, W_out,
           b_out):
    src = edge_index[0]
    dst = edge_index[1]
    pad = EPAD - E
    srcp = jnp.concatenate([src, jnp.arange(pad, dtype=jnp.int32) % N])
    trash = N + jnp.arange(pad, dtype=jnp.int32) % (NPAD - N)
    dstp = jnp.concatenate([dst, trash])
    src3 = srcp.reshape(NW, NCHUNK, CH)
    dst3 = dstp.reshape(NW, NCHUNK, CH)
    xp = jnp.pad(x, ((0, NPAD - N), (0, 0)))
    zh = jnp.zeros((NPAD, D), jnp.float32)
    ones = jnp.ones((CH, D), jnp.float32)

    h0 = _in_proj(xp, W_in, b_in)
    c1 = _count(dst3, zh, ones)
    p1 = _agg(h0, src3, dst3, zh)
    h1, recip = _comb_first(p1, c1)
    p2 = _agg(h1, src3, dst3, zh)
    h = _mix_relu(h1, p2, recip, W_mix0[:H], W_mix0[H:], b_mix0)
    p3 = _agg(h, src3, dst3, zh)
    h1b = _comb(p3, recip)
    p4 = _agg(h1b, src3, dst3, zh)
    out = _mix_out(h1b, p4, recip, W_mix1[:H], W_mix1[H:], b_mix1, W_out,
                   b_out)
    return out[:N]


# speedup vs baseline: 1.0120x; 1.0120x over previous
import functools

import jax
import jax.numpy as jnp
from jax import lax
from jax.experimental import pallas as pl
from jax.experimental.pallas import tpu as pltpu
from jax.experimental.pallas import tpu_sc as plsc

N = 10000
E = 320000
D = 128
H = 128
OUT = 128

NC = 2
NS = 16
NW = NC * NS
CH = 128
NCHUNK = 80
HALF = NCHUNK // 2
PAIRS = HALF // 2
EPAD = NW * NCHUNK * CH
NPAD = 10240
RPT = NPAD // NS
CW = 16

BM = 2560
GRID = NPAD // BM

_mesh = plsc.VectorSubcoreMesh(core_axis_name="c", subcore_axis_name="s")


@functools.partial(
    pl.kernel,
    out_type=jax.ShapeDtypeStruct((NC, NPAD, D), jnp.float32),
    mesh=_mesh,
    scratch_types=[
        pltpu.VMEM((NCHUNK, CH), jnp.int32),
        pltpu.VMEM((CH, D), jnp.float32),
        pltpu.VMEM_SHARED((NPAD, D), jnp.float32),
        pltpu.SemaphoreType.DMA,
        pltpu.SemaphoreType.DMA,
    ],
)
def _count(dst3, zc, ones, c_out, dst_v, ones_v, cacc, sem0, sem1):
    cid = lax.axis_index("c")
    sid = lax.axis_index("s")
    wid = sid * NC + cid
    r0 = sid * RPT
    pltpu.sync_copy(zc.at[pl.ds(r0, RPT)], cacc.at[pl.ds(r0, RPT)])
    pltpu.sync_copy(ones, ones_v)
    pltpu.sync_copy(dst3.at[wid], dst_v)
    plsc.subcore_barrier()

    def step(g, carry):
        j0 = 2 * g
        d0 = pltpu.async_copy(ones_v, cacc.at[dst_v.at[j0]], sem0, add=True)
        d1 = pltpu.async_copy(ones_v, cacc.at[dst_v.at[j0 + 1]], sem1,
                              add=True)
        d0.wait()
        d1.wait()
        return carry

    lax.fori_loop(0, NCHUNK // 2, step, 0)
    plsc.subcore_barrier()
    pltpu.sync_copy(cacc.at[pl.ds(r0, RPT)], c_out.at[cid, pl.ds(r0, RPT)])


@functools.partial(
    pl.kernel,
    out_type=jax.ShapeDtypeStruct((NC, NPAD, D), jnp.float32),
    mesh=_mesh,
    scratch_types=[
        pltpu.VMEM((HALF, CH), jnp.int32),
        pltpu.VMEM((HALF, CH), jnp.int32),
        pltpu.VMEM((CH, D), jnp.float32),
        pltpu.VMEM((CH, D), jnp.float32),
        pltpu.VMEM_SHARED((NPAD, D), jnp.float32),
        pltpu.SemaphoreType.DMA,
        pltpu.SemaphoreType.DMA,
        pltpu.SemaphoreType.DMA,
        pltpu.SemaphoreType.DMA,
    ],
)
def _agg(h, src3, dst3, zh, p_out, src_v, dst_v, rows0, rows1, acc,
         gsem0, gsem1, ssem0, ssem1):
    cid = lax.axis_index("c")
    sid = lax.axis_index("s")
    wid = sid * NC + cid
    r0 = sid * RPT
    pltpu.sync_copy(zh.at[pl.ds(r0, RPT)], acc.at[pl.ds(r0, RPT)])
    plsc.subcore_barrier()

    for half in range(2):
        pltpu.sync_copy(src3.at[wid, pl.ds(half * HALF, HALF)], src_v)
        pltpu.sync_copy(dst3.at[wid, pl.ds(half * HALF, HALF)], dst_v)
        pltpu.async_copy(h.at[src_v.at[0]], rows0, gsem0)
        pltpu.async_copy(h.at[src_v.at[1]], rows1, gsem1)

        def pair(g, carry):
            j0 = 2 * g
            jn0 = jnp.minimum(j0 + 2, HALF - 1)
            jn1 = jnp.minimum(j0 + 3, HALF - 1)
            pltpu.make_async_copy(zh.at[pl.ds(0, CH)], rows0, gsem0).wait()
            s0 = pltpu.async_copy(rows0, acc.at[dst_v.at[j0]], ssem0,
                                  add=True)
            pltpu.make_async_copy(zh.at[pl.ds(0, CH)], rows1, gsem1).wait()
            s1 = pltpu.async_copy(rows1, acc.at[dst_v.at[j0 + 1]], ssem1,
                                  add=True)
            s0.wait()
            pltpu.async_copy(h.at[src_v.at[jn0]], rows0, gsem0)
            s1.wait()
            pltpu.async_copy(h.at[src_v.at[jn1]], rows1, gsem1)
            return carry

        lax.fori_loop(0, PAIRS, pair, 0)
        pltpu.make_async_copy(zh.at[pl.ds(0, CH)], rows0, gsem0).wait()
        pltpu.make_async_copy(zh.at[pl.ds(0, CH)], rows1, gsem1).wait()

    plsc.subcore_barrier()
    pltpu.sync_copy(acc.at[pl.ds(r0, RPT)], p_out.at[cid, pl.ds(r0, RPT)])


def _in_proj_k(x_ref, w_ref, b_ref, o_ref):
    o_ref[...] = jnp.maximum(
        jnp.dot(x_ref[...], w_ref[...], preferred_element_type=jnp.float32)
        + b_ref[...], 0.0)


def _comb_first_k(p_ref, c_ref, h_ref, r_ref):
    cnt = c_ref[0] + c_ref[1]
    r = 1.0 / jnp.maximum(cnt, 1.0)
    r_ref[...] = r[:, 0:CW]
    h_ref[...] = (p_ref[0] + p_ref[1]) * r[:, 0:1]


def _comb_k(p_ref, r_ref, h_ref):
    h_ref[...] = (p_ref[0] + p_ref[1]) * r_ref[:, 0:1]


def _mix_relu_k(h1_ref, p_ref, r_ref, wa_ref, wb_ref, b_ref, o_ref):
    h2 = (p_ref[0] + p_ref[1]) * r_ref[:, 0:1]
    acc = (jnp.dot(h1_ref[...], wa_ref[...], preferred_element_type=jnp.float32)
           + jnp.dot(h2, wb_ref[...], preferred_element_type=jnp.float32)
           + b_ref[...])
    o_ref[...] = jnp.maximum(acc, 0.0)


def _mix_out_k(h1_ref, p_ref, r_ref, wa_ref, wb_ref, b_ref, wo_ref, bo_ref,
               o_ref):
    h2 = (p_ref[0] + p_ref[1]) * r_ref[:, 0:1]
    t = (jnp.dot(h1_ref[...], wa_ref[...], preferred_element_type=jnp.float32)
         + jnp.dot(h2, wb_ref[...], preferred_element_type=jnp.float32)
         + b_ref[...])
    o_ref[...] = (jnp.dot(t, wo_ref[...], preferred_element_type=jnp.float32)
                  + bo_ref[...])


def _rows(i):
    return (i, 0)


def _full2(i):
    return (0, 0)


def _part3(i):
    return (0, i, 0)


_ROWS_D = pl.BlockSpec((BM, D), _rows)
_ROWS_CW = pl.BlockSpec((BM, CW), _rows)
_PART_D = pl.BlockSpec((NC, BM, D), _part3)
_PART_CW = pl.BlockSpec((NC, BM, CW), _part3)
_W_SPEC = pl.BlockSpec((D, H), _full2)
_B_SPEC = pl.BlockSpec((1, H), _full2)


def _in_proj(xp, W, b):
    return pl.pallas_call(
        _in_proj_k,
        grid=(GRID,),
        in_specs=[_ROWS_D, _W_SPEC, _B_SPEC],
        out_specs=_ROWS_D,
        out_shape=jax.ShapeDtypeStruct((NPAD, H), jnp.float32),
    )(xp, W, b.reshape(1, H))


def _comb_first(p, c):
    return pl.pallas_call(
        _comb_first_k,
        grid=(GRID,),
        in_specs=[_PART_D, _PART_D],
        out_specs=[_ROWS_D, _ROWS_CW],
        out_shape=[jax.ShapeDtypeStruct((NPAD, D), jnp.float32),
                   jax.ShapeDtypeStruct((NPAD, CW), jnp.float32)],
    )(p, c)


def _comb(p, recip):
    return pl.pallas_call(
        _comb_k,
        grid=(GRID,),
        in_specs=[_PART_D, _ROWS_CW],
        out_specs=_ROWS_D,
        out_shape=jax.ShapeDtypeStruct((NPAD, D), jnp.float32),
    )(p, recip)


def _mix_relu(h1, p, recip, Wa, Wb, b):
    return pl.pallas_call(
        _mix_relu_k,
        grid=(GRID,),
        in_specs=[_ROWS_D, _PART_D, _ROWS_CW, _W_SPEC, _W_SPEC, _B_SPEC],
        out_specs=_ROWS_D,
        out_shape=jax.ShapeDtypeStruct((NPAD, H), jnp.float32),
    )(h1, p, recip, Wa, Wb, b.reshape(1, H))


def _mix_out(h1, p, recip, Wa, Wb, b, Wo, bo):
    return pl.pallas_call(
        _mix_out_k,
        grid=(GRID,),
        in_specs=[_ROWS_D, _PART_D, _ROWS_CW, _W_SPEC, _W_SPEC, _B_SPEC,
                  _W_SPEC, _B_SPEC],
        out_specs=_ROWS_D,
        out_shape=jax.ShapeDtypeStruct((NPAD, OUT), jnp.float32),
    )(h1, p, recip, Wa, Wb, b.reshape(1, H), Wo, bo.reshape(1, OUT))


def kernel(x, edge_index, W_in, b_in, W_mix0, b_mix0, W_mix1, b_mix1, W_out,
           b_out):
    src = edge_index[0]
    dst = edge_index[1]
    pad = EPAD - E
    srcp = jnp.concatenate([src, jnp.arange(pad, dtype=jnp.int32) % N])
    trash = N + jnp.arange(pad, dtype=jnp.int32) % (NPAD - N)
    dstp = jnp.concatenate([dst, trash])
    src3 = srcp.reshape(NW, NCHUNK, CH)
    dst3 = dstp.reshape(NW, NCHUNK, CH)
    xp = jnp.pad(x, ((0, NPAD - N), (0, 0)))
    zh = jnp.zeros((NPAD, D), jnp.float32)
    ones = jnp.ones((CH, D), jnp.float32)

    h0 = _in_proj(xp, W_in, b_in)
    c1 = _count(dst3, zh, ones)
    p1 = _agg(h0, src3, dst3, zh)
    h1, recip = _comb_first(p1, c1)
    p2 = _agg(h1, src3, dst3, zh)
    h = _mix_relu(h1, p2, recip, W_mix0[:H], W_mix0[H:], b_mix0)
    p3 = _agg(h, src3, dst3, zh)
    h1b = _comb(p3, recip)
    p4 = _agg(h1b, src3, dst3, zh)
    out = _mix_out(h1b, p4, recip, W_mix1[:H], W_mix1[H:], b_mix1, W_out,
                   b_out)
    return out[:N]


# speedup vs baseline: 1.0188x; 1.0067x over previous
import functools

import jax
import jax.numpy as jnp
from jax import lax
from jax.experimental import pallas as pl
from jax.experimental.pallas import tpu as pltpu
from jax.experimental.pallas import tpu_sc as plsc

N = 10000
E = 320000
D = 128
H = 128
OUT = 128

NC = 2
NS = 16
NW = NC * NS
CH = 128
NCHUNK = 80
HALF = NCHUNK // 2
PAIRS = HALF // 2
EPAD = NW * NCHUNK * CH
NPAD = 10240
RPT = NPAD // NS
CW = 16

BM = 2560
GRID = NPAD // BM

_mesh = plsc.VectorSubcoreMesh(core_axis_name="c", subcore_axis_name="s")


@functools.partial(
    pl.kernel,
    out_type=(jax.ShapeDtypeStruct((NC, NPAD, D), jnp.float32),
              jax.ShapeDtypeStruct((NC, NPAD, D), jnp.float32)),
    mesh=_mesh,
    scratch_types=[
        pltpu.VMEM((HALF, CH), jnp.int32),
        pltpu.VMEM((HALF, CH), jnp.int32),
        pltpu.VMEM((CH, D), jnp.float32),
        pltpu.VMEM((CH, D), jnp.float32),
        pltpu.VMEM_SHARED((NPAD, D), jnp.float32),
        pltpu.SemaphoreType.DMA,
        pltpu.SemaphoreType.DMA,
        pltpu.SemaphoreType.DMA,
        pltpu.SemaphoreType.DMA,
    ],
)
def _agg_count(h, src3, dst3, zh, ones, p_out, c_out, src_v, dst_v, rows0,
               rows1, acc, gsem0, gsem1, ssem0, ssem1):
    cid = lax.axis_index("c")
    sid = lax.axis_index("s")
    wid = sid * NC + cid
    r0 = sid * RPT
    pltpu.sync_copy(zh.at[pl.ds(r0, RPT)], acc.at[pl.ds(r0, RPT)])
    plsc.subcore_barrier()

    for half in range(2):
        pltpu.sync_copy(src3.at[wid, pl.ds(half * HALF, HALF)], src_v)
        pltpu.sync_copy(dst3.at[wid, pl.ds(half * HALF, HALF)], dst_v)
        pltpu.async_copy(h.at[src_v.at[0]], rows0, gsem0)
        pltpu.async_copy(h.at[src_v.at[1]], rows1, gsem1)

        def pair(g, carry):
            j0 = 2 * g
            jn0 = jnp.minimum(j0 + 2, HALF - 1)
            jn1 = jnp.minimum(j0 + 3, HALF - 1)
            pltpu.make_async_copy(zh.at[pl.ds(0, CH)], rows0, gsem0).wait()
            s0 = pltpu.async_copy(rows0, acc.at[dst_v.at[j0]], ssem0,
                                  add=True)
            pltpu.make_async_copy(zh.at[pl.ds(0, CH)], rows1, gsem1).wait()
            s1 = pltpu.async_copy(rows1, acc.at[dst_v.at[j0 + 1]], ssem1,
                                  add=True)
            s0.wait()
            pltpu.async_copy(h.at[src_v.at[jn0]], rows0, gsem0)
            s1.wait()
            pltpu.async_copy(h.at[src_v.at[jn1]], rows1, gsem1)
            return carry

        lax.fori_loop(0, PAIRS, pair, 0)
        pltpu.make_async_copy(zh.at[pl.ds(0, CH)], rows0, gsem0).wait()
        pltpu.make_async_copy(zh.at[pl.ds(0, CH)], rows1, gsem1).wait()

    plsc.subcore_barrier()
    pltpu.sync_copy(acc.at[pl.ds(r0, RPT)], p_out.at[cid, pl.ds(r0, RPT)])
    pltpu.sync_copy(zh.at[pl.ds(r0, RPT)], acc.at[pl.ds(r0, RPT)])
    pltpu.sync_copy(ones, rows0)
    plsc.subcore_barrier()

    for half in range(2):
        pltpu.sync_copy(dst3.at[wid, pl.ds(half * HALF, HALF)], dst_v)

        def cstep(g, carry):
            j0 = 2 * g
            d0 = pltpu.async_copy(rows0, acc.at[dst_v.at[j0]], ssem0,
                                  add=True)
            d1 = pltpu.async_copy(rows0, acc.at[dst_v.at[j0 + 1]], ssem1,
                                  add=True)
            d0.wait()
            d1.wait()
            return carry

        lax.fori_loop(0, PAIRS, cstep, 0)

    plsc.subcore_barrier()
    pltpu.sync_copy(acc.at[pl.ds(r0, RPT)], c_out.at[cid, pl.ds(r0, RPT)])


@functools.partial(
    pl.kernel,
    out_type=jax.ShapeDtypeStruct((NC, NPAD, D), jnp.float32),
    mesh=_mesh,
    scratch_types=[
        pltpu.VMEM((HALF, CH), jnp.int32),
        pltpu.VMEM((HALF, CH), jnp.int32),
        pltpu.VMEM((CH, D), jnp.float32),
        pltpu.VMEM((CH, D), jnp.float32),
        pltpu.VMEM_SHARED((NPAD, D), jnp.float32),
        pltpu.SemaphoreType.DMA,
        pltpu.SemaphoreType.DMA,
        pltpu.SemaphoreType.DMA,
        pltpu.SemaphoreType.DMA,
    ],
)
def _agg(h, src3, dst3, zh, p_out, src_v, dst_v, rows0, rows1, acc,
         gsem0, gsem1, ssem0, ssem1):
    cid = lax.axis_index("c")
    sid = lax.axis_index("s")
    wid = sid * NC + cid
    r0 = sid * RPT
    pltpu.sync_copy(zh.at[pl.ds(r0, RPT)], acc.at[pl.ds(r0, RPT)])
    plsc.subcore_barrier()

    for half in range(2):
        pltpu.sync_copy(src3.at[wid, pl.ds(half * HALF, HALF)], src_v)
        pltpu.sync_copy(dst3.at[wid, pl.ds(half * HALF, HALF)], dst_v)
        pltpu.async_copy(h.at[src_v.at[0]], rows0, gsem0)
        pltpu.async_copy(h.at[src_v.at[1]], rows1, gsem1)

        def pair(g, carry):
            j0 = 2 * g
            jn0 = jnp.minimum(j0 + 2, HALF - 1)
            jn1 = jnp.minimum(j0 + 3, HALF - 1)
            pltpu.make_async_copy(zh.at[pl.ds(0, CH)], rows0, gsem0).wait()
            s0 = pltpu.async_copy(rows0, acc.at[dst_v.at[j0]], ssem0,
                                  add=True)
            pltpu.make_async_copy(zh.at[pl.ds(0, CH)], rows1, gsem1).wait()
            s1 = pltpu.async_copy(rows1, acc.at[dst_v.at[j0 + 1]], ssem1,
                                  add=True)
            s0.wait()
            pltpu.async_copy(h.at[src_v.at[jn0]], rows0, gsem0)
            s1.wait()
            pltpu.async_copy(h.at[src_v.at[jn1]], rows1, gsem1)
            return carry

        lax.fori_loop(0, PAIRS, pair, 0)
        pltpu.make_async_copy(zh.at[pl.ds(0, CH)], rows0, gsem0).wait()
        pltpu.make_async_copy(zh.at[pl.ds(0, CH)], rows1, gsem1).wait()

    plsc.subcore_barrier()
    pltpu.sync_copy(acc.at[pl.ds(r0, RPT)], p_out.at[cid, pl.ds(r0, RPT)])


def _in_proj_k(x_ref, w_ref, b_ref, o_ref):
    o_ref[...] = jnp.maximum(
        jnp.dot(x_ref[...], w_ref[...], preferred_element_type=jnp.float32)
        + b_ref[...], 0.0)


def _comb_first_k(p_ref, c_ref, h_ref, r_ref):
    cnt = c_ref[0] + c_ref[1]
    r = 1.0 / jnp.maximum(cnt, 1.0)
    r_ref[...] = r[:, 0:CW]
    h_ref[...] = (p_ref[0] + p_ref[1]) * r[:, 0:1]


def _comb_k(p_ref, r_ref, h_ref):
    h_ref[...] = (p_ref[0] + p_ref[1]) * r_ref[:, 0:1]


def _mix_relu_k(h1_ref, p_ref, r_ref, wa_ref, wb_ref, b_ref, o_ref):
    h2 = (p_ref[0] + p_ref[1]) * r_ref[:, 0:1]
    acc = (jnp.dot(h1_ref[...], wa_ref[...], preferred_element_type=jnp.float32)
           + jnp.dot(h2, wb_ref[...], preferred_element_type=jnp.float32)
           + b_ref[...])
    o_ref[...] = jnp.maximum(acc, 0.0)


def _mix_out_k(h1_ref, p_ref, r_ref, wa_ref, wb_ref, b_ref, wo_ref, bo_ref,
               o_ref):
    h2 = (p_ref[0] + p_ref[1]) * r_ref[:, 0:1]
    t = (jnp.dot(h1_ref[...], wa_ref[...], preferred_element_type=jnp.float32)
         + jnp.dot(h2, wb_ref[...], preferred_element_type=jnp.float32)
         + b_ref[...])
    o_ref[...] = (jnp.dot(t, wo_ref[...], preferred_element_type=jnp.float32)
                  + bo_ref[...])


def _rows(i):
    return (i, 0)


def _full2(i):
    return (0, 0)


def _part3(i):
    return (0, i, 0)


_ROWS_D = pl.BlockSpec((BM, D), _rows)
_ROWS_CW = pl.BlockSpec((BM, CW), _rows)
_PART_D = pl.BlockSpec((NC, BM, D), _part3)
_PART_CW = pl.BlockSpec((NC, BM, CW), _part3)
_W_SPEC = pl.BlockSpec((D, H), _full2)
_B_SPEC = pl.BlockSpec((1, H), _full2)


def _in_proj(xp, W, b):
    return pl.pallas_call(
        _in_proj_k,
        grid=(GRID,),
        in_specs=[_ROWS_D, _W_SPEC, _B_SPEC],
        out_specs=_ROWS_D,
        out_shape=jax.ShapeDtypeStruct((NPAD, H), jnp.float32),
    )(xp, W, b.reshape(1, H))


def _comb_first(p, c):
    return pl.pallas_call(
        _comb_first_k,
        grid=(GRID,),
        in_specs=[_PART_D, _PART_D],
        out_specs=[_ROWS_D, _ROWS_CW],
        out_shape=[jax.ShapeDtypeStruct((NPAD, D), jnp.float32),
                   jax.ShapeDtypeStruct((NPAD, CW), jnp.float32)],
    )(p, c)


def _comb(p, recip):
    return pl.pallas_call(
        _comb_k,
        grid=(GRID,),
        in_specs=[_PART_D, _ROWS_CW],
        out_specs=_ROWS_D,
        out_shape=jax.ShapeDtypeStruct((NPAD, D), jnp.float32),
    )(p, recip)


def _mix_relu(h1, p, recip, Wa, Wb, b):
    return pl.pallas_call(
        _mix_relu_k,
        grid=(GRID,),
        in_specs=[_ROWS_D, _PART_D, _ROWS_CW, _W_SPEC, _W_SPEC, _B_SPEC],
        out_specs=_ROWS_D,
        out_shape=jax.ShapeDtypeStruct((NPAD, H), jnp.float32),
    )(h1, p, recip, Wa, Wb, b.reshape(1, H))


def _mix_out(h1, p, recip, Wa, Wb, b, Wo, bo):
    return pl.pallas_call(
        _mix_out_k,
        grid=(GRID,),
        in_specs=[_ROWS_D, _PART_D, _ROWS_CW, _W_SPEC, _W_SPEC, _B_SPEC,
                  _W_SPEC, _B_SPEC],
        out_specs=_ROWS_D,
        out_shape=jax.ShapeDtypeStruct((NPAD, OUT), jnp.float32),
    )(h1, p, recip, Wa, Wb, b.reshape(1, H), Wo, bo.reshape(1, OUT))


def kernel(x, edge_index, W_in, b_in, W_mix0, b_mix0, W_mix1, b_mix1, W_out,
           b_out):
    src = edge_index[0]
    dst = edge_index[1]
    pad = EPAD - E
    srcp = jnp.concatenate([src, jnp.arange(pad, dtype=jnp.int32) % N])
    trash = N + jnp.arange(pad, dtype=jnp.int32) % (NPAD - N)
    dstp = jnp.concatenate([dst, trash])
    src3 = srcp.reshape(NW, NCHUNK, CH)
    dst3 = dstp.reshape(NW, NCHUNK, CH)
    xp = jnp.pad(x, ((0, NPAD - N), (0, 0)))
    zh = jnp.zeros((NPAD, D), jnp.float32)
    ones = jnp.ones((CH, D), jnp.float32)

    h0 = _in_proj(xp, W_in, b_in)
    p1, c1 = _agg_count(h0, src3, dst3, zh, ones)
    h1, recip = _comb_first(p1, c1)
    p2 = _agg(h1, src3, dst3, zh)
    h = _mix_relu(h1, p2, recip, W_mix0[:H], W_mix0[H:], b_mix0)
    p3 = _agg(h, src3, dst3, zh)
    h1b = _comb(p3, recip)
    p4 = _agg(h1b, src3, dst3, zh)
    out = _mix_out(h1b, p4, recip, W_mix1[:H], W_mix1[H:], b_mix1, W_out,
                   b_out)
    return out[:N]
